# baseline (speedup 1.0000x reference)
import functools

import jax
import jax.numpy as jnp
from jax import lax
from jax.experimental import pallas as pl
from jax.experimental.pallas import tpu as pltpu

NEG_BIG = -1e30


def _router_body(x_ref, gw_ref, gb_ref, w_ref, loss_ref):
    x = x_ref[:]
    gw = gw_ref[:]
    logits = lax.dot_general(x, gw, (((1,), (1,)), ((), ())),
                             preferred_element_type=jnp.float32)
    logits = logits + gb_ref[:]
    T, EP = logits.shape
    m = jnp.max(logits, axis=1, keepdims=True)
    p = jnp.exp(logits - m)
    probs = p / jnp.sum(p, axis=1, keepdims=True)

    lane = lax.broadcasted_iota(jnp.int32, (T, EP), 1)
    m1 = jnp.max(probs, axis=1, keepdims=True)
    a1 = jnp.min(jnp.where(probs == m1, lane, EP), axis=1, keepdims=True)
    probs_wo1 = jnp.where(lane == a1, -1.0, probs)
    m2 = jnp.max(probs_wo1, axis=1, keepdims=True)
    a2 = jnp.min(jnp.where(probs_wo1 == m2, lane, EP), axis=1, keepdims=True)
    w = jnp.where(lane == a1, m1, 0.0) + jnp.where(lane == a2, m2, 0.0)
    w_ref[:] = w[:, :w_ref.shape[1]]

    usage = jnp.sum(probs, axis=0, keepdims=True) / T
    ul = usage * jnp.log(usage + 1e-9)
    ul = jnp.where(lax.broadcasted_iota(jnp.int32, (1, EP), 1) < w_ref.shape[1],
                   ul, 0.0)
    loss_ref[0, 0] = jnp.sum(ul)


def _expert_body(x_ref, w1_ref, b1_ref, w2_ref, b2_ref, w3_ref, b3_ref,
                 wc_ref, out_ref, acc_ref):
    e = pl.program_id(1)
    j = pl.program_id(2)
    nj = pl.num_programs(2)

    x = x_ref[:].astype(jnp.bfloat16)
    w1 = w1_ref[0].astype(jnp.bfloat16)
    w2 = w2_ref[0].astype(jnp.bfloat16)
    h = lax.dot_general(x, w1, (((1,), (1,)), ((), ())),
                        preferred_element_type=jnp.float32) + b1_ref[0]
    g = lax.dot_general(x, w2, (((1,), (1,)), ((), ())),
                        preferred_element_type=jnp.float32) + b2_ref[0]
    hg = (h * jax.nn.sigmoid(g)).astype(jnp.bfloat16)
    w3 = w3_ref[0].astype(jnp.bfloat16)
    part = lax.dot_general(hg, w3, (((1,), (1,)), ((), ())),
                           preferred_element_type=jnp.float32)

    @pl.when(j == 0)
    def _():
        acc_ref[:] = part

    @pl.when(j > 0)
    def _():
        acc_ref[:] = acc_ref[:] + part

    @pl.when(jnp.logical_and(e == 0, j == 0))
    def _():
        out_ref[:] = jnp.zeros_like(out_ref)

    @pl.when(j == nj - 1)
    def _():
        wc = wc_ref[:]
        lane = lax.broadcasted_iota(jnp.int32, wc.shape, 1)
        col = jnp.sum(jnp.where(lane == e, wc, 0.0), axis=1, keepdims=True)
        out_ref[:] = out_ref[:] + col * (acc_ref[:] + b3_ref[0])


def kernel(x, gate_W, gate_b, w1_W, w1_b, w2_W, w2_b, w3_W, w3_b):
    B, S, DIM = x.shape
    E, HIDDEN = w1_b.shape
    T = B * S
    xf = x.reshape(T, DIM)

    EP = 128
    gw_p = jnp.zeros((EP, DIM), jnp.float32).at[:E].set(gate_W)
    gb_p = jnp.full((1, EP), NEG_BIG, jnp.float32).at[0, :E].set(gate_b)

    wc, loss2 = pl.pallas_call(
        _router_body,
        out_shape=(
            jax.ShapeDtypeStruct((T, E), jnp.float32),
            jax.ShapeDtypeStruct((1, 1), jnp.float32),
        ),
        in_specs=[
            pl.BlockSpec((T, DIM), lambda: (0, 0)),
            pl.BlockSpec((EP, DIM), lambda: (0, 0)),
            pl.BlockSpec((1, EP), lambda: (0, 0)),
        ],
        out_specs=(
            pl.BlockSpec((T, E), lambda: (0, 0)),
            pl.BlockSpec(memory_space=pltpu.SMEM),
        ),
    )(xf, gw_p, gb_p)

    HB = min(256, HIDDEN)
    TM = min(2048, T)
    nj = HIDDEN // HB
    nm = T // TM
    grid = (nm, E, nj)
    out = pl.pallas_call(
        _expert_body,
        grid=grid,
        out_shape=jax.ShapeDtypeStruct((T, DIM), jnp.float32),
        in_specs=[
            pl.BlockSpec((TM, DIM), lambda m, e, j: (m, 0)),
            pl.BlockSpec((1, HB, DIM), lambda m, e, j: (e, j, 0)),
            pl.BlockSpec((1, 1, HB), lambda m, e, j: (e, 0, j)),
            pl.BlockSpec((1, HB, DIM), lambda m, e, j: (e, j, 0)),
            pl.BlockSpec((1, 1, HB), lambda m, e, j: (e, 0, j)),
            pl.BlockSpec((1, DIM, HB), lambda m, e, j: (e, 0, j)),
            pl.BlockSpec((1, 1, DIM), lambda m, e, j: (e, 0, 0)),
            pl.BlockSpec((TM, E), lambda m, e, j: (m, 0)),
        ],
        out_specs=pl.BlockSpec((TM, DIM), lambda m, e, j: (m, 0)),
        scratch_shapes=[pltpu.VMEM((TM, DIM), jnp.float32)],
    )(xf, w1_W, w1_b.reshape(E, 1, HIDDEN), w2_W, w2_b.reshape(E, 1, HIDDEN),
      w3_W, w3_b.reshape(E, 1, DIM), wc)

    return out.reshape(B, S, DIM), loss2[0, 0]


# speedup vs baseline: 1.4731x; 1.4731x over previous
import functools

import jax
import jax.numpy as jnp
from jax import lax
from jax.experimental import pallas as pl
from jax.experimental.pallas import tpu as pltpu
from jax.experimental.pallas import tpu_sc as plsc

NEG_BIG = -1e30


def _router_body(tn, x_ref, gw_ref, gb_ref,
                 pos0_ref, pos1_ref, q0_ref, q1_ref, off_ref, loss_ref):
    x = x_ref[:]
    gw = gw_ref[:]
    logits = lax.dot_general(x, gw, (((1,), (1,)), ((), ())),
                             preferred_element_type=jnp.float32)
    logits = logits + gb_ref[:]
    T, EP = logits.shape
    E = off_ref.shape[1] // 2
    m = jnp.max(logits, axis=1, keepdims=True)
    p = jnp.exp(logits - m)
    probs = p / jnp.sum(p, axis=1, keepdims=True)

    lane = lax.broadcasted_iota(jnp.int32, (T, EP), 1)
    m1 = jnp.max(probs, axis=1, keepdims=True)
    a1 = jnp.min(jnp.where(probs == m1, lane, EP), axis=1, keepdims=True)
    probs_wo1 = jnp.where(lane == a1, -1.0, probs)
    m2 = jnp.max(probs_wo1, axis=1, keepdims=True)
    a2 = jnp.min(jnp.where(probs_wo1 == m2, lane, EP), axis=1, keepdims=True)
    wc = jnp.where(lane == a1, m1, 0.0) + jnp.where(lane == a2, m2, 0.0)
    wc = wc[:, :E]

    usage = jnp.sum(probs, axis=0, keepdims=True) / T
    ul = usage * jnp.log(usage + 1e-9)
    ul = jnp.where(lax.broadcasted_iota(jnp.int32, (1, EP), 1) < E, ul, 0.0)
    loss_ref[0, 0] = jnp.sum(ul)

    lane_e = lax.broadcasted_iota(jnp.int32, (T, E), 1)
    c = (wc > 0.0).astype(jnp.int32)
    cum = c
    s0 = 1
    while s0 < T:
        cum = cum + jnp.concatenate(
            [jnp.zeros((s0, E), jnp.int32), cum[:T - s0]], axis=0)
        s0 *= 2
    counts = cum[T - 1:T, :]
    tiles = (counts + tn - 1) // tn
    incl = tiles
    s = 1
    while s < E:
        incl = incl + jnp.concatenate(
            [jnp.zeros((1, s), jnp.int32), incl[:, :E - s]], axis=1)
        s *= 2
    off_tiles = incl - tiles
    off_ref[:] = jnp.concatenate(
        [off_tiles, jnp.full((1, E), 10 ** 6, jnp.int32)], axis=1)

    dstrow = off_tiles * tn + cum - 1
    e0 = jnp.min(jnp.where(c > 0, lane_e, E), axis=1, keepdims=True)
    e1 = jnp.max(jnp.where(c > 0, lane_e, -1), axis=1, keepdims=True)
    pos0_ref[:] = jnp.sum(jnp.where(lane_e == e0, dstrow, 0), axis=1,
                          keepdims=True)
    pos1_ref[:] = jnp.sum(jnp.where(lane_e == e1, dstrow, 0), axis=1,
                          keepdims=True)
    q0v = jnp.sum(jnp.where(lane_e == e0, wc, 0.0), axis=1, keepdims=True)
    q1v = jnp.sum(jnp.where(lane_e == e1, wc, 0.0), axis=1, keepdims=True)
    lane128 = lax.broadcasted_iota(jnp.int32, (T, 128), 1)
    q0_ref[:] = jnp.where(lane128 == 0, q0v, 0.0)
    q1_ref[:] = jnp.where(lane128 == 0, q1v, 0.0)


def _run_router(xf, gate_W, gate_b, T, DIM, E, TN):
    EP = 128
    gw_p = jnp.zeros((EP, DIM), jnp.float32).at[:E].set(gate_W)
    gb_p = jnp.full((1, EP), NEG_BIG, jnp.float32).at[0, :E].set(gate_b)
    return pl.pallas_call(
        functools.partial(_router_body, TN),
        out_shape=(
            jax.ShapeDtypeStruct((T, 1), jnp.int32),
            jax.ShapeDtypeStruct((T, 1), jnp.int32),
            jax.ShapeDtypeStruct((T, 128), jnp.float32),
            jax.ShapeDtypeStruct((T, 128), jnp.float32),
            jax.ShapeDtypeStruct((1, 2 * E), jnp.int32),
            jax.ShapeDtypeStruct((1, 1), jnp.float32),
        ),
        in_specs=[
            pl.BlockSpec((T, DIM), lambda: (0, 0)),
            pl.BlockSpec((EP, DIM), lambda: (0, 0)),
            pl.BlockSpec((1, EP), lambda: (0, 0)),
        ],
        out_specs=(
            pl.BlockSpec((T, 1), lambda: (0, 0)),
            pl.BlockSpec((T, 1), lambda: (0, 0)),
            pl.BlockSpec((T, 128), lambda: (0, 0)),
            pl.BlockSpec((T, 128), lambda: (0, 0)),
            pl.BlockSpec((1, 2 * E), lambda: (0, 0)),
            pl.BlockSpec(memory_space=pltpu.SMEM),
        ),
    )(xf, gw_p, gb_p)


def _dispatch_sc(xf, pos0, pos1, q0, q1, padrows):
    T, DIM = xf.shape
    NW = 32
    per_w = T // NW
    C = 64 if per_w % 64 == 0 else per_w
    nchunk = per_w // C
    mesh = plsc.VectorSubcoreMesh(core_axis_name="c", subcore_axis_name="s")

    @functools.partial(
        pl.kernel, mesh=mesh,
        out_type=(
            jax.ShapeDtypeStruct((padrows, DIM), jnp.float32),
            jax.ShapeDtypeStruct((padrows, 128), jnp.float32),
        ),
        scratch_types=[
            pltpu.VMEM((C, DIM), jnp.float32),
            pltpu.VMEM((C,), jnp.int32),
            pltpu.VMEM((C,), jnp.int32),
            pltpu.VMEM((C, 128), jnp.float32),
            pltpu.SemaphoreType.DMA,
        ],
    )
    def k(x_hbm, p0_hbm, p1_hbm, q0_hbm, q1_hbm, xs_hbm, wr_hbm,
          buf, idx0, idx1, qbuf, sem):
        wid = lax.axis_index("s") * 2 + lax.axis_index("c")
        for ch in range(nchunk):
            base = wid * per_w + ch * C
            pltpu.sync_copy(p0_hbm.at[pl.ds(base, C)], idx0)
            pltpu.sync_copy(p1_hbm.at[pl.ds(base, C)], idx1)
            pltpu.sync_copy(x_hbm.at[pl.ds(base, C)], buf)
            pltpu.async_copy(buf, xs_hbm.at[idx0], sem).wait()
            pltpu.async_copy(buf, xs_hbm.at[idx1], sem).wait()
            pltpu.sync_copy(q0_hbm.at[pl.ds(base, C)], qbuf)
            pltpu.async_copy(qbuf, wr_hbm.at[idx0], sem).wait()
            pltpu.sync_copy(q1_hbm.at[pl.ds(base, C)], qbuf)
            pltpu.async_copy(qbuf, wr_hbm.at[idx1], sem).wait()

    return k(xf, pos0, pos1, q0, q1)


def _grouped_body(nj, nd, hb, off_ref, xs_ref, w1_ref, b1_ref, w2_ref, b2_ref,
                  w3_ref, b3_ref, wr_ref, ys_ref, xsbf_ref, hg_ref):
    j = pl.program_id(1)

    @pl.when(j == 0)
    def _():
        xsbf_ref[:] = xs_ref[:].astype(jnp.bfloat16)

    @pl.when(j < nj)
    def _():
        xsbf = xsbf_ref[:]
        w1 = w1_ref[0].astype(jnp.bfloat16)
        w2 = w2_ref[0].astype(jnp.bfloat16)
        h = lax.dot_general(w1, xsbf, (((1,), (1,)), ((), ())),
                            preferred_element_type=jnp.float32)
        g = lax.dot_general(w2, xsbf, (((1,), (1,)), ((), ())),
                            preferred_element_type=jnp.float32)
        h = h + b1_ref[0]
        g = g + b2_ref[0]
        hg = h * jax.nn.sigmoid(g)
        hg_ref[pl.ds(j * hb, hb), :] = hg.astype(jnp.bfloat16)

    @pl.when(j >= nj)
    def _():
        w3 = w3_ref[0].astype(jnp.bfloat16)
        part = lax.dot_general(hg_ref[:], w3, (((0,), (1,)), ((), ())),
                               preferred_element_type=jnp.float32)
        ys_ref[:] = (part + b3_ref[0]) * wr_ref[:, 0:1]


def _run_grouped(xs, wrow, w1_W, w1_b, w2_W, w2_b, w3_W, w3_b,
                 off, NT, TN, E):
    PADROWS, DIM = xs.shape
    _, HIDDEN = w1_b.shape
    HB = min(256, HIDDEN)
    DB = min(512, DIM)
    nj = HIDDEN // HB
    nd = DIM // DB

    def eid(i, off_ref):
        s = jnp.zeros((), jnp.int32)
        for e in range(E):
            s += jnp.where(i >= off_ref[e], 1, 0).astype(jnp.int32)
        return s - 1

    def jc(j):
        return jnp.minimum(j, nj - 1)

    def dc(j):
        return jnp.maximum(j - nj, 0)

    grid_spec = pltpu.PrefetchScalarGridSpec(
        num_scalar_prefetch=1,
        grid=(NT, nj + nd),
        in_specs=[
            pl.BlockSpec((TN, DIM), lambda i, j, o: (i, 0)),
            pl.BlockSpec((1, HB, DIM), lambda i, j, o: (eid(i, o), jc(j), 0)),
            pl.BlockSpec((1, HB, 1), lambda i, j, o: (eid(i, o), jc(j), 0)),
            pl.BlockSpec((1, HB, DIM), lambda i, j, o: (eid(i, o), jc(j), 0)),
            pl.BlockSpec((1, HB, 1), lambda i, j, o: (eid(i, o), jc(j), 0)),
            pl.BlockSpec((1, DB, HIDDEN), lambda i, j, o: (eid(i, o), dc(j), 0)),
            pl.BlockSpec((1, 1, DB), lambda i, j, o: (eid(i, o), 0, dc(j))),
            pl.BlockSpec((TN, 128), lambda i, j, o: (i, 0)),
        ],
        out_specs=pl.BlockSpec((TN, DB), lambda i, j, o: (i, dc(j))),
        scratch_shapes=[
            pltpu.VMEM((TN, DIM), jnp.bfloat16),
            pltpu.VMEM((HIDDEN, TN), jnp.bfloat16),
        ],
    )
    return pl.pallas_call(
        functools.partial(_grouped_body, nj, nd, HB),
        grid_spec=grid_spec,
        out_shape=jax.ShapeDtypeStruct((PADROWS, DIM), jnp.float32),
    )(off, xs, w1_W, w1_b.reshape(E, HIDDEN, 1), w2_W,
      w2_b.reshape(E, HIDDEN, 1), w3_W, w3_b.reshape(E, 1, DIM), wrow)


def _combine_sc(ysw, pos0, pos1, T):
    PADROWS, DIM = ysw.shape
    NW = 32
    per_w = T // NW
    C = 32 if per_w % 32 == 0 else per_w
    nchunk = per_w // C
    nvec = DIM // 16
    mesh = plsc.VectorSubcoreMesh(core_axis_name="c", subcore_axis_name="s")

    @functools.partial(
        pl.kernel, mesh=mesh,
        out_type=jax.ShapeDtypeStruct((T, DIM), jnp.float32),
        scratch_types=[
            pltpu.VMEM((C, DIM), jnp.float32),
            pltpu.VMEM((C, DIM), jnp.float32),
            pltpu.VMEM((C,), jnp.int32),
            pltpu.VMEM((C,), jnp.int32),
            pltpu.SemaphoreType.DMA,
        ],
    )
    def k(ys_hbm, p0_hbm, p1_hbm, out_hbm, y0, y1, idx0, idx1, sem):
        wid = lax.axis_index("s") * 2 + lax.axis_index("c")
        for ch in range(nchunk):
            base = wid * per_w + ch * C
            pltpu.sync_copy(p0_hbm.at[pl.ds(base, C)], idx0)
            pltpu.sync_copy(p1_hbm.at[pl.ds(base, C)], idx1)
            pltpu.async_copy(ys_hbm.at[idx0], y0, sem).wait()
            pltpu.async_copy(ys_hbm.at[idx1], y1, sem).wait()

            def row(r, _):
                def col(v, _):
                    sl = pl.ds(v * 16, 16)
                    y0[r, sl] = y0[r, sl] + y1[r, sl]
                    return 0
                lax.fori_loop(0, nvec, col, 0)
                return 0
            lax.fori_loop(0, C, row, 0)
            pltpu.sync_copy(y0, out_hbm.at[pl.ds(base, C)])

    return k(ysw, pos0, pos1)


def kernel(x, gate_W, gate_b, w1_W, w1_b, w2_W, w2_b, w3_W, w3_b):
    B, S, DIM = x.shape
    E, HIDDEN = w1_b.shape
    T = B * S
    K = 2
    xf = x.reshape(T, DIM)

    TN = min(512, T)
    NT = (K * T) // TN + E - 1
    PADROWS = NT * TN

    pos0, pos1, q0, q1, off, loss2 = _run_router(
        xf, gate_W, gate_b, T, DIM, E, TN)
    p0f = pos0.reshape(T)
    p1f = pos1.reshape(T)

    xs, wrow = _dispatch_sc(xf, p0f, p1f, q0, q1, PADROWS)
    ysw = _run_grouped(xs, wrow, w1_W, w1_b, w2_W, w2_b, w3_W, w3_b,
                       off.reshape(2 * E), NT, TN, E)
    out = _combine_sc(ysw, p0f, p1f, T)
    return out.reshape(B, S, DIM), loss2[0, 0]


# speedup vs baseline: 1.5876x; 1.0778x over previous
import functools

import jax
import jax.numpy as jnp
from jax import lax
from jax.experimental import pallas as pl
from jax.experimental.pallas import tpu as pltpu
from jax.experimental.pallas import tpu_sc as plsc

NEG_BIG = -1e30


def _router_body(tn, x_ref, gw_ref, gb_ref,
                 pos0_ref, pos1_ref, q0_ref, q1_ref, off_ref, loss_ref):
    x = x_ref[:]
    gw = gw_ref[:]
    logits = lax.dot_general(x, gw, (((1,), (1,)), ((), ())),
                             preferred_element_type=jnp.float32)
    logits = logits + gb_ref[:]
    T, EP = logits.shape
    E = off_ref.shape[1] // 2
    m = jnp.max(logits, axis=1, keepdims=True)
    p = jnp.exp(logits - m)
    probs = p / jnp.sum(p, axis=1, keepdims=True)

    lane = lax.broadcasted_iota(jnp.int32, (T, EP), 1)
    m1 = jnp.max(probs, axis=1, keepdims=True)
    a1 = jnp.min(jnp.where(probs == m1, lane, EP), axis=1, keepdims=True)
    probs_wo1 = jnp.where(lane == a1, -1.0, probs)
    m2 = jnp.max(probs_wo1, axis=1, keepdims=True)
    a2 = jnp.min(jnp.where(probs_wo1 == m2, lane, EP), axis=1, keepdims=True)
    wc = jnp.where(lane == a1, m1, 0.0) + jnp.where(lane == a2, m2, 0.0)
    wc = wc[:, :E]

    usage = jnp.sum(probs, axis=0, keepdims=True) / T
    ul = usage * jnp.log(usage + 1e-9)
    ul = jnp.where(lax.broadcasted_iota(jnp.int32, (1, EP), 1) < E, ul, 0.0)
    loss_ref[0, 0] = jnp.sum(ul)

    lane_e = lax.broadcasted_iota(jnp.int32, (T, E), 1)
    c = (wc > 0.0).astype(jnp.int32)
    cum = c
    s0 = 1
    while s0 < T:
        cum = cum + jnp.concatenate(
            [jnp.zeros((s0, E), jnp.int32), cum[:T - s0]], axis=0)
        s0 *= 2
    counts = cum[T - 1:T, :]
    tiles = (counts + tn - 1) // tn
    incl = tiles
    s = 1
    while s < E:
        incl = incl + jnp.concatenate(
            [jnp.zeros((1, s), jnp.int32), incl[:, :E - s]], axis=1)
        s *= 2
    off_tiles = incl - tiles
    off_ref[:] = jnp.concatenate(
        [off_tiles, jnp.full((1, E), 10 ** 6, jnp.int32)], axis=1)

    dstrow = off_tiles * tn + cum - 1
    e0 = jnp.min(jnp.where(c > 0, lane_e, E), axis=1, keepdims=True)
    e1 = jnp.max(jnp.where(c > 0, lane_e, -1), axis=1, keepdims=True)
    pos0_ref[:] = jnp.sum(jnp.where(lane_e == e0, dstrow, 0), axis=1,
                          keepdims=True)
    pos1_ref[:] = jnp.sum(jnp.where(lane_e == e1, dstrow, 0), axis=1,
                          keepdims=True)
    q0v = jnp.sum(jnp.where(lane_e == e0, wc, 0.0), axis=1, keepdims=True)
    q1v = jnp.sum(jnp.where(lane_e == e1, wc, 0.0), axis=1, keepdims=True)
    lane128 = lax.broadcasted_iota(jnp.int32, (T, 128), 1)
    q0_ref[:] = jnp.where(lane128 == 0, q0v, 0.0)
    q1_ref[:] = jnp.where(lane128 == 0, q1v, 0.0)


def _run_router(xf, gate_W, gate_b, T, DIM, E, TN):
    EP = 128
    gw_p = jnp.zeros((EP, DIM), jnp.float32).at[:E].set(gate_W)
    gb_p = jnp.full((1, EP), NEG_BIG, jnp.float32).at[0, :E].set(gate_b)
    return pl.pallas_call(
        functools.partial(_router_body, TN),
        out_shape=(
            jax.ShapeDtypeStruct((T, 1), jnp.int32),
            jax.ShapeDtypeStruct((T, 1), jnp.int32),
            jax.ShapeDtypeStruct((T, 128), jnp.float32),
            jax.ShapeDtypeStruct((T, 128), jnp.float32),
            jax.ShapeDtypeStruct((1, 2 * E), jnp.int32),
            jax.ShapeDtypeStruct((1, 1), jnp.float32),
        ),
        in_specs=[
            pl.BlockSpec((T, DIM), lambda: (0, 0)),
            pl.BlockSpec((EP, DIM), lambda: (0, 0)),
            pl.BlockSpec((1, EP), lambda: (0, 0)),
        ],
        out_specs=(
            pl.BlockSpec((T, 1), lambda: (0, 0)),
            pl.BlockSpec((T, 1), lambda: (0, 0)),
            pl.BlockSpec((T, 128), lambda: (0, 0)),
            pl.BlockSpec((T, 128), lambda: (0, 0)),
            pl.BlockSpec((1, 2 * E), lambda: (0, 0)),
            pl.BlockSpec(memory_space=pltpu.SMEM),
        ),
    )(xf, gw_p, gb_p)


def _dispatch_sc(xf, pos0, pos1, q0, q1, padrows):
    T, DIM = xf.shape
    NW = 32
    per_w = T // NW
    C = 64 if per_w % 64 == 0 else per_w
    nchunk = per_w // C
    mesh = plsc.VectorSubcoreMesh(core_axis_name="c", subcore_axis_name="s")

    @functools.partial(
        pl.kernel, mesh=mesh,
        out_type=(
            jax.ShapeDtypeStruct((padrows, DIM), jnp.float32),
            jax.ShapeDtypeStruct((padrows, 128), jnp.float32),
        ),
        scratch_types=[
            pltpu.VMEM((C, DIM), jnp.float32),
            pltpu.VMEM((C,), jnp.int32),
            pltpu.VMEM((C,), jnp.int32),
            pltpu.VMEM((C, 128), jnp.float32),
            pltpu.SemaphoreType.DMA,
        ],
    )
    def k(x_hbm, p0_hbm, p1_hbm, q0_hbm, q1_hbm, xs_hbm, wr_hbm,
          buf, idx0, idx1, qbuf, sem):
        wid = lax.axis_index("s") * 2 + lax.axis_index("c")
        for ch in range(nchunk):
            base = wid * per_w + ch * C
            pltpu.sync_copy(p0_hbm.at[pl.ds(base, C)], idx0)
            pltpu.sync_copy(p1_hbm.at[pl.ds(base, C)], idx1)
            pltpu.sync_copy(x_hbm.at[pl.ds(base, C)], buf)
            pltpu.async_copy(buf, xs_hbm.at[idx0], sem).wait()
            pltpu.async_copy(buf, xs_hbm.at[idx1], sem).wait()
            pltpu.sync_copy(q0_hbm.at[pl.ds(base, C)], qbuf)
            pltpu.async_copy(qbuf, wr_hbm.at[idx0], sem).wait()
            pltpu.sync_copy(q1_hbm.at[pl.ds(base, C)], qbuf)
            pltpu.async_copy(qbuf, wr_hbm.at[idx1], sem).wait()

    return k(xf, pos0, pos1, q0, q1)


def _grouped_body(nj, nd, hb, off_ref, xs_ref, w1_ref, b1_ref, w2_ref, b2_ref,
                  w3_ref, b3_ref, wr_ref, ys_ref, xsbf_ref, hg_ref):
    j = pl.program_id(1)

    @pl.when(j == 0)
    def _():
        xsbf_ref[:] = xs_ref[:].astype(jnp.bfloat16)

    @pl.when(j < nj)
    def _():
        xsbf = xsbf_ref[:]
        w1 = w1_ref[0].astype(jnp.bfloat16)
        w2 = w2_ref[0].astype(jnp.bfloat16)
        h = lax.dot_general(xsbf, w1, (((1,), (1,)), ((), ())),
                            preferred_element_type=jnp.float32)
        g = lax.dot_general(xsbf, w2, (((1,), (1,)), ((), ())),
                            preferred_element_type=jnp.float32)
        h = h + b1_ref[0]
        g = g + b2_ref[0]
        hg = h * jax.nn.sigmoid(g)
        hg_ref[:, pl.ds(j * hb, hb)] = hg.astype(jnp.bfloat16)

    @pl.when(j >= nj)
    def _():
        w3 = w3_ref[0].astype(jnp.bfloat16)
        part = lax.dot_general(hg_ref[:], w3, (((1,), (1,)), ((), ())),
                               preferred_element_type=jnp.float32)
        ys_ref[:] = (part + b3_ref[0]) * wr_ref[:, 0:1]


def _run_grouped(xs, wrow, w1_W, w1_b, w2_W, w2_b, w3_W, w3_b,
                 off, NT, TN, E):
    PADROWS, DIM = xs.shape
    _, HIDDEN = w1_b.shape
    HB = min(256, HIDDEN)
    DB = DIM
    nj = HIDDEN // HB
    nd = DIM // DB

    def eid(i, off_ref):
        s = jnp.zeros((), jnp.int32)
        for e in range(E):
            s += jnp.where(i >= off_ref[e], 1, 0).astype(jnp.int32)
        return s - 1

    def jc(j):
        return jnp.minimum(j, nj - 1)

    def dc(j):
        return jnp.maximum(j - nj, 0)

    grid_spec = pltpu.PrefetchScalarGridSpec(
        num_scalar_prefetch=1,
        grid=(NT, nj + nd),
        in_specs=[
            pl.BlockSpec((TN, DIM), lambda i, j, o: (i, 0)),
            pl.BlockSpec((1, HB, DIM), lambda i, j, o: (eid(i, o), jc(j), 0)),
            pl.BlockSpec((1, 1, HB), lambda i, j, o: (eid(i, o), 0, jc(j))),
            pl.BlockSpec((1, HB, DIM), lambda i, j, o: (eid(i, o), jc(j), 0)),
            pl.BlockSpec((1, 1, HB), lambda i, j, o: (eid(i, o), 0, jc(j))),
            pl.BlockSpec((1, DB, HIDDEN), lambda i, j, o: (eid(i, o), dc(j), 0)),
            pl.BlockSpec((1, 1, DB), lambda i, j, o: (eid(i, o), 0, dc(j))),
            pl.BlockSpec((TN, 128), lambda i, j, o: (i, 0)),
        ],
        out_specs=pl.BlockSpec((TN, DB), lambda i, j, o: (i, dc(j))),
        scratch_shapes=[
            pltpu.VMEM((TN, DIM), jnp.bfloat16),
            pltpu.VMEM((TN, HIDDEN), jnp.bfloat16),
        ],
    )
    return pl.pallas_call(
        functools.partial(_grouped_body, nj, nd, HB),
        grid_spec=grid_spec,
        out_shape=jax.ShapeDtypeStruct((PADROWS, DIM), jnp.float32),
    )(off, xs, w1_W, w1_b.reshape(E, 1, HIDDEN), w2_W,
      w2_b.reshape(E, 1, HIDDEN), w3_W, w3_b.reshape(E, 1, DIM), wrow)


def _combine_sc(ysw, pos0, pos1, T):
    PADROWS, DIM = ysw.shape
    NW = 32
    per_w = T // NW
    C = 32 if per_w % 32 == 0 else per_w
    nchunk = per_w // C
    nvec = DIM // 16
    mesh = plsc.VectorSubcoreMesh(core_axis_name="c", subcore_axis_name="s")

    @functools.partial(
        pl.kernel, mesh=mesh,
        out_type=jax.ShapeDtypeStruct((T, DIM), jnp.float32),
        scratch_types=[
            pltpu.VMEM((C, DIM), jnp.float32),
            pltpu.VMEM((C, DIM), jnp.float32),
            pltpu.VMEM((C,), jnp.int32),
            pltpu.VMEM((C,), jnp.int32),
            pltpu.SemaphoreType.DMA,
        ],
    )
    def k(ys_hbm, p0_hbm, p1_hbm, out_hbm, y0, y1, idx0, idx1, sem):
        wid = lax.axis_index("s") * 2 + lax.axis_index("c")
        for ch in range(nchunk):
            base = wid * per_w + ch * C
            pltpu.sync_copy(p0_hbm.at[pl.ds(base, C)], idx0)
            pltpu.sync_copy(p1_hbm.at[pl.ds(base, C)], idx1)
            pltpu.async_copy(ys_hbm.at[idx0], y0, sem).wait()
            pltpu.async_copy(ys_hbm.at[idx1], y1, sem).wait()

            def row(r, _):
                def col(v, _):
                    sl = pl.ds(v * 16, 16)
                    y0[r, sl] = y0[r, sl] + y1[r, sl]
                    return 0
                lax.fori_loop(0, nvec, col, 0)
                return 0
            lax.fori_loop(0, C, row, 0)
            pltpu.sync_copy(y0, out_hbm.at[pl.ds(base, C)])

    return k(ysw, pos0, pos1)


def kernel(x, gate_W, gate_b, w1_W, w1_b, w2_W, w2_b, w3_W, w3_b):
    B, S, DIM = x.shape
    E, HIDDEN = w1_b.shape
    T = B * S
    K = 2
    xf = x.reshape(T, DIM)

    TN = min(512, T)
    NT = (K * T) // TN + E - 1
    PADROWS = NT * TN

    pos0, pos1, q0, q1, off, loss2 = _run_router(
        xf, gate_W, gate_b, T, DIM, E, TN)
    p0f = pos0.reshape(T)
    p1f = pos1.reshape(T)

    xs, wrow = _dispatch_sc(xf, p0f, p1f, q0, q1, PADROWS)
    ysw = _run_grouped(xs, wrow, w1_W, w1_b, w2_W, w2_b, w3_W, w3_b,
                       off.reshape(2 * E), NT, TN, E)
    out = _combine_sc(ysw, p0f, p1f, T)
    return out.reshape(B, S, DIM), loss2[0, 0]


# speedup vs baseline: 1.7693x; 1.1144x over previous
import functools

import jax
import jax.numpy as jnp
from jax import lax
from jax.experimental import pallas as pl
from jax.experimental.pallas import tpu as pltpu
from jax.experimental.pallas import tpu_sc as plsc

NEG_BIG = -1e30


def _router_body(tn, x_ref, gw_ref, gb_ref,
                 pos0_ref, pos1_ref, q0_ref, q1_ref, off_ref, loss_ref):
    x = x_ref[:]
    gw = gw_ref[:]
    logits = lax.dot_general(x, gw, (((1,), (1,)), ((), ())),
                             preferred_element_type=jnp.float32)
    logits = logits + gb_ref[:]
    T, EP = logits.shape
    E = off_ref.shape[1] // 2
    m = jnp.max(logits, axis=1, keepdims=True)
    p = jnp.exp(logits - m)
    probs = p / jnp.sum(p, axis=1, keepdims=True)

    lane = lax.broadcasted_iota(jnp.int32, (T, EP), 1)
    m1 = jnp.max(probs, axis=1, keepdims=True)
    a1 = jnp.min(jnp.where(probs == m1, lane, EP), axis=1, keepdims=True)
    probs_wo1 = jnp.where(lane == a1, -1.0, probs)
    m2 = jnp.max(probs_wo1, axis=1, keepdims=True)
    a2 = jnp.min(jnp.where(probs_wo1 == m2, lane, EP), axis=1, keepdims=True)
    wc = jnp.where(lane == a1, m1, 0.0) + jnp.where(lane == a2, m2, 0.0)
    wc = wc[:, :E]

    usage = jnp.sum(probs, axis=0, keepdims=True) / T
    ul = usage * jnp.log(usage + 1e-9)
    ul = jnp.where(lax.broadcasted_iota(jnp.int32, (1, EP), 1) < E, ul, 0.0)
    loss_ref[0, 0] = jnp.sum(ul)

    lane_e = lax.broadcasted_iota(jnp.int32, (T, E), 1)
    c = (wc > 0.0).astype(jnp.int32)
    cum = c
    s0 = 1
    while s0 < T:
        cum = cum + jnp.concatenate(
            [jnp.zeros((s0, E), jnp.int32), cum[:T - s0]], axis=0)
        s0 *= 2
    counts = cum[T - 1:T, :]
    tiles = (counts + tn - 1) // tn
    incl = tiles
    s = 1
    while s < E:
        incl = incl + jnp.concatenate(
            [jnp.zeros((1, s), jnp.int32), incl[:, :E - s]], axis=1)
        s *= 2
    off_tiles = incl - tiles
    off_ref[:] = jnp.concatenate(
        [off_tiles, jnp.full((1, E), 10 ** 6, jnp.int32)], axis=1)

    dstrow = off_tiles * tn + cum - 1
    e0 = jnp.min(jnp.where(c > 0, lane_e, E), axis=1, keepdims=True)
    e1 = jnp.max(jnp.where(c > 0, lane_e, -1), axis=1, keepdims=True)
    pos0_ref[:] = jnp.sum(jnp.where(lane_e == e0, dstrow, 0), axis=1,
                          keepdims=True)
    pos1_ref[:] = jnp.sum(jnp.where(lane_e == e1, dstrow, 0), axis=1,
                          keepdims=True)
    q0v = jnp.sum(jnp.where(lane_e == e0, wc, 0.0), axis=1, keepdims=True)
    q1v = jnp.sum(jnp.where(lane_e == e1, wc, 0.0), axis=1, keepdims=True)
    lane128 = lax.broadcasted_iota(jnp.int32, (T, 128), 1)
    q0_ref[:] = jnp.where(lane128 == 0, q0v, 0.0)
    q1_ref[:] = jnp.where(lane128 == 0, q1v, 0.0)


def _run_router(xf, gate_W, gate_b, T, DIM, E, TN):
    EP = 128
    gw_p = jnp.zeros((EP, DIM), jnp.float32).at[:E].set(gate_W)
    gb_p = jnp.full((1, EP), NEG_BIG, jnp.float32).at[0, :E].set(gate_b)
    return pl.pallas_call(
        functools.partial(_router_body, TN),
        out_shape=(
            jax.ShapeDtypeStruct((T, 1), jnp.int32),
            jax.ShapeDtypeStruct((T, 1), jnp.int32),
            jax.ShapeDtypeStruct((T, 128), jnp.float32),
            jax.ShapeDtypeStruct((T, 128), jnp.float32),
            jax.ShapeDtypeStruct((1, 2 * E), jnp.int32),
            jax.ShapeDtypeStruct((1, 1), jnp.float32),
        ),
        in_specs=[
            pl.BlockSpec((T, DIM), lambda: (0, 0)),
            pl.BlockSpec((EP, DIM), lambda: (0, 0)),
            pl.BlockSpec((1, EP), lambda: (0, 0)),
        ],
        out_specs=(
            pl.BlockSpec((T, 1), lambda: (0, 0)),
            pl.BlockSpec((T, 1), lambda: (0, 0)),
            pl.BlockSpec((T, 128), lambda: (0, 0)),
            pl.BlockSpec((T, 128), lambda: (0, 0)),
            pl.BlockSpec((1, 2 * E), lambda: (0, 0)),
            pl.BlockSpec(memory_space=pltpu.SMEM),
        ),
    )(xf, gw_p, gb_p)


def _dispatch_sc(xf, pos0, pos1, q0, q1, padrows):
    T, DIM = xf.shape
    NW = 32
    per_w = T // NW
    C = 64 if per_w % 64 == 0 else per_w
    nchunk = per_w // C
    mesh = plsc.VectorSubcoreMesh(core_axis_name="c", subcore_axis_name="s")

    @functools.partial(
        pl.kernel, mesh=mesh,
        out_type=(
            jax.ShapeDtypeStruct((padrows, DIM), jnp.float32),
            jax.ShapeDtypeStruct((padrows, 128), jnp.float32),
        ),
        scratch_types=[
            pltpu.VMEM((C, DIM), jnp.float32),
            pltpu.VMEM((C,), jnp.int32),
            pltpu.VMEM((C,), jnp.int32),
            pltpu.VMEM((C, 128), jnp.float32),
            pltpu.SemaphoreType.DMA,
        ],
    )
    def k(x_hbm, p0_hbm, p1_hbm, q0_hbm, q1_hbm, xs_hbm, wr_hbm,
          buf, idx0, idx1, qbuf, sem):
        wid = lax.axis_index("s") * 2 + lax.axis_index("c")
        for ch in range(nchunk):
            base = wid * per_w + ch * C
            pltpu.sync_copy(p0_hbm.at[pl.ds(base, C)], idx0)
            pltpu.sync_copy(p1_hbm.at[pl.ds(base, C)], idx1)
            pltpu.sync_copy(x_hbm.at[pl.ds(base, C)], buf)
            pltpu.async_copy(buf, xs_hbm.at[idx0], sem).wait()
            pltpu.async_copy(buf, xs_hbm.at[idx1], sem).wait()
            pltpu.sync_copy(q0_hbm.at[pl.ds(base, C)], qbuf)
            pltpu.async_copy(qbuf, wr_hbm.at[idx0], sem).wait()
            pltpu.sync_copy(q1_hbm.at[pl.ds(base, C)], qbuf)
            pltpu.async_copy(qbuf, wr_hbm.at[idx1], sem).wait()

    return k(xf, pos0, pos1, q0, q1)


def _grouped_body(nj, nd, hb, off_ref, xs_ref, w1_ref, b1_ref, w2_ref, b2_ref,
                  w3_ref, b3_ref, wr_ref, ys_ref, xsbf_ref, hg_ref):
    j = pl.program_id(1)

    @pl.when(j == 0)
    def _():
        xsbf_ref[:] = xs_ref[:].astype(jnp.bfloat16)

    @pl.when(j < nj)
    def _():
        xsbf = xsbf_ref[:]
        w1 = w1_ref[0].astype(jnp.bfloat16)
        w2 = w2_ref[0].astype(jnp.bfloat16)
        h = lax.dot_general(xsbf, w1, (((1,), (1,)), ((), ())),
                            preferred_element_type=jnp.float32)
        g = lax.dot_general(xsbf, w2, (((1,), (1,)), ((), ())),
                            preferred_element_type=jnp.float32)
        h = h + b1_ref[0]
        g = g + b2_ref[0]
        hg = h * jax.nn.sigmoid(g)
        hg_ref[:, pl.ds(j * hb, hb)] = hg.astype(jnp.bfloat16)

    @pl.when(j >= nj)
    def _():
        w3 = w3_ref[0].astype(jnp.bfloat16)
        part = lax.dot_general(hg_ref[:], w3, (((1,), (1,)), ((), ())),
                               preferred_element_type=jnp.float32)
        ys_ref[:] = (part + b3_ref[0]) * wr_ref[:, 0:1]


def _run_grouped(xs, wrow, w1_W, w1_b, w2_W, w2_b, w3_W, w3_b,
                 off, NT, TN, E):
    PADROWS, DIM = xs.shape
    _, HIDDEN = w1_b.shape
    HB = min(512, HIDDEN)
    DB = min(512, DIM)
    nj = HIDDEN // HB
    nd = DIM // DB

    def eid(i, off_ref):
        s = jnp.zeros((), jnp.int32)
        for e in range(E):
            s += jnp.where(i >= off_ref[e], 1, 0).astype(jnp.int32)
        return s - 1

    def jc(j):
        return jnp.minimum(j, nj - 1)

    def dc(j):
        return jnp.maximum(j - nj, 0)

    grid_spec = pltpu.PrefetchScalarGridSpec(
        num_scalar_prefetch=1,
        grid=(NT, nj + nd),
        in_specs=[
            pl.BlockSpec((TN, DIM), lambda i, j, o: (i, 0)),
            pl.BlockSpec((1, HB, DIM), lambda i, j, o: (eid(i, o), jc(j), 0)),
            pl.BlockSpec((1, 1, HB), lambda i, j, o: (eid(i, o), 0, jc(j))),
            pl.BlockSpec((1, HB, DIM), lambda i, j, o: (eid(i, o), jc(j), 0)),
            pl.BlockSpec((1, 1, HB), lambda i, j, o: (eid(i, o), 0, jc(j))),
            pl.BlockSpec((1, DB, HIDDEN), lambda i, j, o: (eid(i, o), dc(j), 0)),
            pl.BlockSpec((1, 1, DB), lambda i, j, o: (eid(i, o), 0, dc(j))),
            pl.BlockSpec((TN, 128), lambda i, j, o: (i, 0)),
        ],
        out_specs=pl.BlockSpec((TN, DB), lambda i, j, o: (i, dc(j))),
        scratch_shapes=[
            pltpu.VMEM((TN, DIM), jnp.bfloat16),
            pltpu.VMEM((TN, HIDDEN), jnp.bfloat16),
        ],
    )
    return pl.pallas_call(
        functools.partial(_grouped_body, nj, nd, HB),
        grid_spec=grid_spec,
        out_shape=jax.ShapeDtypeStruct((PADROWS, DIM), jnp.float32),
    )(off, xs, w1_W, w1_b.reshape(E, 1, HIDDEN), w2_W,
      w2_b.reshape(E, 1, HIDDEN), w3_W, w3_b.reshape(E, 1, DIM), wrow)


def _combine_sc(ysw, pos0, pos1, T):
    PADROWS, DIM = ysw.shape
    NW = 32
    per_w = T // NW
    C = 32 if per_w % 32 == 0 else per_w
    nchunk = per_w // C
    nvec = DIM // 16
    mesh = plsc.VectorSubcoreMesh(core_axis_name="c", subcore_axis_name="s")

    @functools.partial(
        pl.kernel, mesh=mesh,
        out_type=jax.ShapeDtypeStruct((T, DIM), jnp.float32),
        scratch_types=[
            pltpu.VMEM((C, DIM), jnp.float32),
            pltpu.VMEM((C, DIM), jnp.float32),
            pltpu.VMEM((C,), jnp.int32),
            pltpu.VMEM((C,), jnp.int32),
            pltpu.SemaphoreType.DMA,
        ],
    )
    def k(ys_hbm, p0_hbm, p1_hbm, out_hbm, y0, y1, idx0, idx1, sem):
        wid = lax.axis_index("s") * 2 + lax.axis_index("c")
        for ch in range(nchunk):
            base = wid * per_w + ch * C
            pltpu.sync_copy(p0_hbm.at[pl.ds(base, C)], idx0)
            pltpu.sync_copy(p1_hbm.at[pl.ds(base, C)], idx1)
            pltpu.async_copy(ys_hbm.at[idx0], y0, sem).wait()
            pltpu.async_copy(ys_hbm.at[idx1], y1, sem).wait()

            def row(r, _):
                def col(v, _):
                    sl = pl.ds(v * 16, 16)
                    y0[r, sl] = y0[r, sl] + y1[r, sl]
                    return 0
                lax.fori_loop(0, nvec, col, 0)
                return 0
            lax.fori_loop(0, C, row, 0)
            pltpu.sync_copy(y0, out_hbm.at[pl.ds(base, C)])

    return k(ysw, pos0, pos1)


def kernel(x, gate_W, gate_b, w1_W, w1_b, w2_W, w2_b, w3_W, w3_b):
    B, S, DIM = x.shape
    E, HIDDEN = w1_b.shape
    T = B * S
    K = 2
    xf = x.reshape(T, DIM)

    TN = min(512, T)
    NT = (K * T) // TN + E - 1
    PADROWS = NT * TN

    pos0, pos1, q0, q1, off, loss2 = _run_router(
        xf, gate_W, gate_b, T, DIM, E, TN)
    p0f = pos0.reshape(T)
    p1f = pos1.reshape(T)

    xs, wrow = _dispatch_sc(xf, p0f, p1f, q0, q1, PADROWS)
    ysw = _run_grouped(xs, wrow, w1_W, w1_b, w2_W, w2_b, w3_W, w3_b,
                       off.reshape(2 * E), NT, TN, E)
    out = _combine_sc(ysw, p0f, p1f, T)
    return out.reshape(B, S, DIM), loss2[0, 0]
